# baseline (speedup 1.0000x reference)
import jax
import jax.numpy as jnp
from jax.experimental import pallas as pl
from jax.experimental.pallas import tpu as pltpu

_STATE_SPACE_SIZE = 10.0
_ACTION_SPACE_SIZE = 5.0
_N_SAMPLES = 2048
_TAIL = 4
_FAR = 64
_PUSH = 4

_BIG = 3.0e38

_BI = 8
_TILE_N = 1024


def _space_samples(state_dim, action_dim):
    key_s, key_a = jax.random.split(jax.random.key(42))
    s = (jax.random.uniform(key_s, (_N_SAMPLES, state_dim),
                            dtype=jnp.float32) * 2 - 1) * _STATE_SPACE_SIZE
    a = (jax.random.uniform(key_a, (_N_SAMPLES, action_dim),
                            dtype=jnp.float32) * 2 - 1) * _ACTION_SPACE_SIZE
    return s, a


def _dist_accum(k_dim, bt_ref, s_ref, d_ref):
    n = bt_ref.shape[1]

    def strip(h, carry):
        base = h * _TILE_N
        part0 = jnp.zeros((_BI, _TILE_N), jnp.float32)
        part1 = jnp.zeros((_BI, _TILE_N), jnp.float32)
        for k in range(0, k_dim, 2):
            srow0 = s_ref[:, k:k + 1]
            brow0 = bt_ref[k * _BI:k * _BI + _BI, pl.ds(base, _TILE_N)]
            part0 = part0 + jnp.abs(srow0 - brow0)
            srow1 = s_ref[:, k + 1:k + 2]
            brow1 = bt_ref[(k + 1) * _BI:(k + 2) * _BI, pl.ds(base, _TILE_N)]
            part1 = part1 + jnp.abs(srow1 - brow1)
        d_ref[:, pl.ds(base, _TILE_N)] = part0 + part1
        return carry

    jax.lax.fori_loop(0, n // _TILE_N, strip, 0)


def _min4_extract(d, s4_ref, row0):
    n = d.shape[1]
    lane = jax.lax.broadcasted_iota(jnp.int32, (_BI, n), 1)
    for t in range(_TAIL):
        m = jnp.min(d, axis=1, keepdims=True)
        cand = jnp.where(d == m, lane, n)
        idx = jnp.min(cand, axis=1, keepdims=True)
        d = jnp.where(lane == idx, _BIG, d)
        s4_ref[pl.ds(row0, _BI), t:t + 1] = m


def _top64_sq_sum(s4):
    tail = jnp.sum(s4, axis=1, keepdims=True)
    sq = jnp.sum(s4 * s4, axis=1, keepdims=True)
    ridx = jax.lax.broadcasted_iota(jnp.int32, tail.shape, 0)

    def step(_, carry):
        t, acc = carry
        m = jnp.max(t)
        cand = jnp.where(t == m, ridx, _N_SAMPLES)
        cidx = jnp.min(cand)
        hit = ridx == cidx
        acc = acc + jnp.sum(jnp.where(hit, sq, 0.0))
        t = jnp.where(hit, -_BIG, t)
        return t, acc

    _, acc = jax.lax.fori_loop(0, _FAR, step, (tail, jnp.float32(0.0)))
    return acc


def _coverage_call(s_samples, a_samples, ls, la):
    ks, ka = ls.shape[1], la.shape[1]
    n = _N_SAMPLES
    nb = _N_SAMPLES // _BI

    def bcast8(bt, k_dim):
        b = jnp.broadcast_to(bt[:, None, :], (k_dim, _BI, n))
        return b.reshape(k_dim * _BI, n)

    def body(bts_ref, ss_ref, bta_ref, sa_ref, ls_ref, la_ref, out_ref,
             ds_ref, da_ref, s4s_ref, s4a_ref):
        i = pl.program_id(0)
        _dist_accum(ks, bts_ref, ss_ref, ds_ref)
        _dist_accum(ka, bta_ref, sa_ref, da_ref)
        _min4_extract(ds_ref[:, :], s4s_ref, i * _BI)
        _min4_extract(da_ref[:, :], s4a_ref, i * _BI)

        @pl.when(i == nb - 1)
        def _():
            s_norm = jnp.sum(jnp.abs(ls_ref[:, :]), axis=1)
            a_norm = jnp.sum(jnp.abs(la_ref[:, :]), axis=1)
            s_viol = jnp.maximum(s_norm - _STATE_SPACE_SIZE, 0.0)
            a_viol = jnp.maximum(a_norm - _ACTION_SPACE_SIZE, 0.0)
            size_loss = jnp.mean(s_viol * s_viol) + jnp.mean(a_viol * a_viol)

            s_cov = _top64_sq_sum(s4s_ref[:, :]) / (_FAR * _PUSH)
            a_cov = _top64_sq_sum(s4a_ref[:, :]) / (_FAR * _PUSH)
            out_ref[:, :] = jnp.broadcast_to(size_loss + s_cov + a_cov, (1, 1))

    return pl.pallas_call(
        body,
        grid=(nb,),
        in_specs=[
            pl.BlockSpec((ks * _BI, n), lambda i: (0, 0)),
            pl.BlockSpec((_BI, ks), lambda i: (i, 0)),
            pl.BlockSpec((ka * _BI, n), lambda i: (0, 0)),
            pl.BlockSpec((_BI, ka), lambda i: (i, 0)),
            pl.BlockSpec((n, ks), lambda i: (0, 0)),
            pl.BlockSpec((n, ka), lambda i: (0, 0)),
        ],
        out_specs=pl.BlockSpec((1, 1), lambda i: (0, 0)),
        out_shape=jax.ShapeDtypeStruct((1, 1), jnp.float32),
        scratch_shapes=[
            pltpu.VMEM((_BI, n), jnp.float32),
            pltpu.VMEM((_BI, n), jnp.float32),
            pltpu.VMEM((n, _TAIL), jnp.float32),
            pltpu.VMEM((n, _TAIL), jnp.float32),
        ],
        compiler_params=pltpu.CompilerParams(
            dimension_semantics=("arbitrary",),
        ),
    )(bcast8(ls.T, ks), s_samples, bcast8(la.T, ka), a_samples, ls, la)


@jax.jit
def kernel(latent_states, latent_actions):
    ls = latent_states.reshape(-1, latent_states.shape[-1])
    la = latent_actions.reshape(-1, latent_actions.shape[-1])
    s_samples, a_samples = _space_samples(ls.shape[-1], la.shape[-1])
    out = _coverage_call(s_samples, a_samples, ls, la)
    return out[0, 0]


# speedup vs baseline: 1.0849x; 1.0849x over previous
import jax
import jax.numpy as jnp
from jax.experimental import pallas as pl
from jax.experimental.pallas import tpu as pltpu

_STATE_SPACE_SIZE = 10.0
_ACTION_SPACE_SIZE = 5.0
_N_SAMPLES = 2048
_TAIL = 4
_FAR = 64
_PUSH = 4

_BIG = 3.0e38

_BI = 8
_TILE_N = 1024


def _space_samples(state_dim, action_dim):
    key_s, key_a = jax.random.split(jax.random.key(42))
    s = (jax.random.uniform(key_s, (_N_SAMPLES, state_dim),
                            dtype=jnp.float32) * 2 - 1) * _STATE_SPACE_SIZE
    a = (jax.random.uniform(key_a, (_N_SAMPLES, action_dim),
                            dtype=jnp.float32) * 2 - 1) * _ACTION_SPACE_SIZE
    return s, a


def _dist_accum(k_dim, bt_ref, s_ref, d_ref):
    n = bt_ref.shape[1]

    def strip(h, carry):
        base = h * _TILE_N
        part0 = jnp.zeros((_BI, _TILE_N), jnp.float32)
        part1 = jnp.zeros((_BI, _TILE_N), jnp.float32)
        for k in range(0, k_dim, 2):
            srow0 = s_ref[:, k:k + 1]
            brow0 = bt_ref[k * _BI:k * _BI + _BI, pl.ds(base, _TILE_N)]
            part0 = part0 + jnp.abs(srow0 - brow0)
            srow1 = s_ref[:, k + 1:k + 2]
            brow1 = bt_ref[(k + 1) * _BI:(k + 2) * _BI, pl.ds(base, _TILE_N)]
            part1 = part1 + jnp.abs(srow1 - brow1)
        d_ref[:, pl.ds(base, _TILE_N)] = part0 + part1
        return carry

    jax.lax.fori_loop(0, n // _TILE_N, strip, 0)


def _min4_extract(d, s4_ref, row0):
    n = d.shape[1]
    lane = jax.lax.broadcasted_iota(jnp.int32, (_BI, n), 1)
    for t in range(_TAIL):
        m = jnp.min(d, axis=1, keepdims=True)
        cand = jnp.where(d == m, lane, n)
        idx = jnp.min(cand, axis=1, keepdims=True)
        d = jnp.where(lane == idx, _BIG, d)
        s4_ref[pl.ds(row0, _BI), t:t + 1] = m


def _top64_sq_sum(s4):
    tail = jnp.sum(s4, axis=1)
    sq = jnp.sum(s4 * s4, axis=1)
    tail = jnp.reshape(tail, (_N_SAMPLES // 128, 128))
    sq = jnp.reshape(sq, (_N_SAMPLES // 128, 128))
    ridx = jax.lax.broadcasted_iota(jnp.int32, tail.shape, 0) * 128 + \
        jax.lax.broadcasted_iota(jnp.int32, tail.shape, 1)

    def step(_, carry):
        t, acc = carry
        m = jnp.max(t)
        cand = jnp.where(t == m, ridx, _N_SAMPLES)
        cidx = jnp.min(cand)
        hit = ridx == cidx
        acc = acc + jnp.sum(jnp.where(hit, sq, 0.0))
        t = jnp.where(hit, -_BIG, t)
        return t, acc

    _, acc = jax.lax.fori_loop(0, _FAR, step, (tail, jnp.float32(0.0)))
    return acc


def _coverage_call(s_samples, a_samples, ls, la):
    ks, ka = ls.shape[1], la.shape[1]
    n = _N_SAMPLES
    nb = _N_SAMPLES // _BI

    def bcast8(bt, k_dim):
        b = jnp.broadcast_to(bt[:, None, :], (k_dim, _BI, n))
        return b.reshape(k_dim * _BI, n)

    def body(bts_ref, ss_ref, bta_ref, sa_ref, ls_ref, la_ref, out_ref,
             ds_ref, da_ref, s4s_ref, s4a_ref):
        i = pl.program_id(0)
        _dist_accum(ks, bts_ref, ss_ref, ds_ref)
        _dist_accum(ka, bta_ref, sa_ref, da_ref)
        _min4_extract(ds_ref[:, :], s4s_ref, i * _BI)
        _min4_extract(da_ref[:, :], s4a_ref, i * _BI)

        @pl.when(i == nb - 1)
        def _():
            s_norm = jnp.sum(jnp.abs(ls_ref[:, :]), axis=1)
            a_norm = jnp.sum(jnp.abs(la_ref[:, :]), axis=1)
            s_viol = jnp.maximum(s_norm - _STATE_SPACE_SIZE, 0.0)
            a_viol = jnp.maximum(a_norm - _ACTION_SPACE_SIZE, 0.0)
            size_loss = jnp.mean(s_viol * s_viol) + jnp.mean(a_viol * a_viol)

            s_cov = _top64_sq_sum(s4s_ref[:, :]) / (_FAR * _PUSH)
            a_cov = _top64_sq_sum(s4a_ref[:, :]) / (_FAR * _PUSH)
            out_ref[:, :] = jnp.broadcast_to(size_loss + s_cov + a_cov, (1, 1))

    return pl.pallas_call(
        body,
        grid=(nb,),
        in_specs=[
            pl.BlockSpec((ks * _BI, n), lambda i: (0, 0)),
            pl.BlockSpec((_BI, ks), lambda i: (i, 0)),
            pl.BlockSpec((ka * _BI, n), lambda i: (0, 0)),
            pl.BlockSpec((_BI, ka), lambda i: (i, 0)),
            pl.BlockSpec((n, ks), lambda i: (0, 0)),
            pl.BlockSpec((n, ka), lambda i: (0, 0)),
        ],
        out_specs=pl.BlockSpec((1, 1), lambda i: (0, 0)),
        out_shape=jax.ShapeDtypeStruct((1, 1), jnp.float32),
        scratch_shapes=[
            pltpu.VMEM((_BI, n), jnp.float32),
            pltpu.VMEM((_BI, n), jnp.float32),
            pltpu.VMEM((n, _TAIL), jnp.float32),
            pltpu.VMEM((n, _TAIL), jnp.float32),
        ],
        compiler_params=pltpu.CompilerParams(
            dimension_semantics=("arbitrary",),
        ),
    )(bcast8(ls.T, ks), s_samples, bcast8(la.T, ka), a_samples, ls, la)


@jax.jit
def kernel(latent_states, latent_actions):
    ls = latent_states.reshape(-1, latent_states.shape[-1])
    la = latent_actions.reshape(-1, latent_actions.shape[-1])
    s_samples, a_samples = _space_samples(ls.shape[-1], la.shape[-1])
    out = _coverage_call(s_samples, a_samples, ls, la)
    return out[0, 0]
